# baseline (device time: 189184 ns/iter reference)
import jax
import jax.numpy as jnp
from jax import lax
from jax.experimental import pallas as pl
from jax.experimental.pallas import tpu as pltpu

N_DEV = 16
M = 2048
N = 2048
CHUNK = M // N_DEV
N_SUB = 8
SUB = CHUNK // N_SUB
H = 2 * (N_DEV - 1)
S = 4


def kernel(x, w_mat):
    def body(x_ref, w_ref, out_ref, *scratch):
        comms = scratch[0:N_SUB]
        sems = scratch[N_SUB:3 * N_SUB]
        credits = scratch[3 * N_SUB:4 * N_SUB]

        me = lax.axis_index("i")
        left = (me + N_DEV - 1) % N_DEV
        right = (me + 1) % N_DEV

        out_ref[:, :] = jnp.dot(
            x_ref[:, :], w_ref[:, :], preferred_element_type=jnp.float32
        )

        barrier_sem = pltpu.get_barrier_semaphore()
        for nbr in (left, right):
            pl.semaphore_signal(
                barrier_sem, inc=1,
                device_id=(nbr,), device_id_type=pl.DeviceIdType.MESH,
            )
        pl.semaphore_wait(barrier_sem, 2)

        rings = [
            (comms[r], sems[2 * r], sems[2 * r + 1], credits[r],
             right if r % 2 == 0 else left,
             left if r % 2 == 0 else right,
             r % 2 == 0)
            for r in range(N_SUB)
        ]

        def rows(c, r):
            return pl.ds(c * CHUNK + r * SUB, SUB)

        def chunk_at(h, fwd):
            if fwd:
                return (me + 3 * N_DEV - 1 - h) % N_DEV
            return (me + 1 + h) % N_DEV

        rdmas = []
        for comm, s_sems, r_sems, _, dst, _, _ in rings:
            rdmas.append([
                pltpu.make_async_remote_copy(
                    src_ref=comm.at[h % S],
                    dst_ref=comm.at[(h + 1) % S],
                    send_sem=s_sems.at[h % S],
                    recv_sem=r_sems.at[(h + 1) % S],
                    device_id=(dst,),
                    device_id_type=pl.DeviceIdType.MESH,
                )
                for h in range(H)
            ])

        for r, (comm, *_rest) in enumerate(rings):
            comm[0, :, :] = out_ref[rows(me, r), :]

        for h in range(H):
            slot = h % S
            for r, (comm, _, _, credit, _, credit_dst, fwd) in enumerate(rings):
                if h >= 1:
                    rdmas[r][h - 1].wait_recv()
                    if h <= N_DEV - 1:
                        c = chunk_at(h - 1, fwd)
                        comm[slot, :, :] = (
                            comm[slot, :, :] + out_ref[rows(c, r), :]
                        )
                if h >= 2:
                    rdmas[r][h - 2].wait_send()
                    if h <= H - 2:
                        pl.semaphore_signal(
                            credit, inc=1,
                            device_id=(credit_dst,),
                            device_id_type=pl.DeviceIdType.MESH,
                        )
                if h >= 3:
                    pl.semaphore_wait(credit, 1)
                rdmas[r][h].start()
                if h == N_DEV - 1:
                    c = chunk_at(h - 1, fwd)
                    out_ref[rows(c, r), :] = comm[slot, :, :]
                elif h >= N_DEV:
                    a = h - N_DEV
                    c = (me + 2 * N_DEV - a) % N_DEV if fwd else (me + a) % N_DEV
                    out_ref[rows(c, r), :] = comm[slot, :, :]

        slot = H % S
        for r, (comm, _, _, _, _, _, fwd) in enumerate(rings):
            rdmas[r][H - 1].wait_recv()
            c = (me + N_DEV + 2) % N_DEV if fwd else (me + N_DEV - 2) % N_DEV
            out_ref[rows(c, r), :] = comm[slot, :, :]
        for r in range(len(rings)):
            rdmas[r][H - 2].wait_send()
            rdmas[r][H - 1].wait_send()

    scratch_shapes = (
        [pltpu.VMEM((S, SUB, N), jnp.float32) for _ in range(N_SUB)]
        + [pltpu.SemaphoreType.DMA((S,)) for _ in range(2 * N_SUB)]
        + [pltpu.SemaphoreType.REGULAR for _ in range(N_SUB)]
    )
    return pl.pallas_call(
        body,
        out_shape=jax.ShapeDtypeStruct((M, N), jnp.float32),
        in_specs=[
            pl.BlockSpec(memory_space=pltpu.VMEM),
            pl.BlockSpec(memory_space=pltpu.VMEM),
        ],
        out_specs=pl.BlockSpec(memory_space=pltpu.VMEM),
        scratch_shapes=scratch_shapes,
        compiler_params=pltpu.CompilerParams(collective_id=0),
    )(x, w_mat)


# device time: 185643 ns/iter; 1.0191x vs baseline; 1.0191x over previous
import jax
import jax.numpy as jnp
from jax import lax
from jax.experimental import pallas as pl
from jax.experimental.pallas import tpu as pltpu

N_DEV = 16
M = 2048
N = 2048
CHUNK = M // N_DEV
N_SUB = 4
SUB = CHUNK // N_SUB
H = 2 * (N_DEV - 1)
S = 4


def kernel(x, w_mat):
    def body(x_ref, w_ref, out_ref, *scratch):
        comms = scratch[0:N_SUB]
        sems = scratch[N_SUB:3 * N_SUB]
        credits = scratch[3 * N_SUB:4 * N_SUB]

        me = lax.axis_index("i")
        left = (me + N_DEV - 1) % N_DEV
        right = (me + 1) % N_DEV

        out_ref[pl.ds(me * CHUNK, CHUNK), :] = jnp.dot(
            x_ref[pl.ds(me * CHUNK, CHUNK), :], w_ref[:, :],
            preferred_element_type=jnp.float32,
        )

        barrier_sem = pltpu.get_barrier_semaphore()
        for nbr in (left, right):
            pl.semaphore_signal(
                barrier_sem, inc=1,
                device_id=(nbr,), device_id_type=pl.DeviceIdType.MESH,
            )
        pl.semaphore_wait(barrier_sem, 2)

        rings = [
            (comms[r], sems[2 * r], sems[2 * r + 1], credits[r],
             right if r % 2 == 0 else left,
             left if r % 2 == 0 else right,
             r % 2 == 0)
            for r in range(N_SUB)
        ]

        def rows(c, r):
            return pl.ds(c * CHUNK + r * SUB, SUB)

        def chunk_at(h, fwd):
            if fwd:
                return (me + 3 * N_DEV - 1 - h) % N_DEV
            return (me + 1 + h) % N_DEV

        rdmas = []
        for comm, s_sems, r_sems, _, dst, _, _ in rings:
            rdmas.append([
                pltpu.make_async_remote_copy(
                    src_ref=comm.at[h % S],
                    dst_ref=comm.at[(h + 1) % S],
                    send_sem=s_sems.at[h % S],
                    recv_sem=r_sems.at[(h + 1) % S],
                    device_id=(dst,),
                    device_id_type=pl.DeviceIdType.MESH,
                )
                for h in range(H)
            ])

        for r, (comm, *_rest) in enumerate(rings):
            comm[0, :, :] = out_ref[rows(me, r), :]
        for r in range(N_SUB):
            rdmas[r][0].start()
        out_ref[:, :] = jnp.dot(
            x_ref[:, :], w_ref[:, :], preferred_element_type=jnp.float32
        )

        for h in range(1, H):
            slot = h % S
            for r, (comm, _, _, credit, _, credit_dst, fwd) in enumerate(rings):
                if h >= 2:
                    rdmas[r][h - 2].wait_send()
                    if h <= H - 2:
                        pl.semaphore_signal(
                            credit, inc=1,
                            device_id=(credit_dst,),
                            device_id_type=pl.DeviceIdType.MESH,
                        )
                rdmas[r][h - 1].wait_recv()
                if h <= N_DEV - 1:
                    c = chunk_at(h - 1, fwd)
                    comm[slot, :, :] = (
                        comm[slot, :, :] + out_ref[rows(c, r), :]
                    )
                if h >= 3:
                    pl.semaphore_wait(credit, 1)
                rdmas[r][h].start()
                if h == N_DEV - 1:
                    c = chunk_at(h - 1, fwd)
                    out_ref[rows(c, r), :] = comm[slot, :, :]
                elif h >= N_DEV:
                    a = h - N_DEV
                    c = (me + 2 * N_DEV - a) % N_DEV if fwd else (me + a) % N_DEV
                    out_ref[rows(c, r), :] = comm[slot, :, :]

        slot = H % S
        for r, (comm, _, _, _, _, _, fwd) in enumerate(rings):
            rdmas[r][H - 1].wait_recv()
            c = (me + N_DEV + 2) % N_DEV if fwd else (me + N_DEV - 2) % N_DEV
            out_ref[rows(c, r), :] = comm[slot, :, :]
        for r in range(len(rings)):
            rdmas[r][H - 2].wait_send()
            rdmas[r][H - 1].wait_send()

    scratch_shapes = (
        [pltpu.VMEM((S, SUB, N), jnp.float32) for _ in range(N_SUB)]
        + [pltpu.SemaphoreType.DMA((S,)) for _ in range(2 * N_SUB)]
        + [pltpu.SemaphoreType.REGULAR for _ in range(N_SUB)]
    )
    return pl.pallas_call(
        body,
        out_shape=jax.ShapeDtypeStruct((M, N), jnp.float32),
        in_specs=[
            pl.BlockSpec(memory_space=pltpu.VMEM),
            pl.BlockSpec(memory_space=pltpu.VMEM),
        ],
        out_specs=pl.BlockSpec(memory_space=pltpu.VMEM),
        scratch_shapes=scratch_shapes,
        compiler_params=pltpu.CompilerParams(collective_id=0),
    )(x, w_mat)
